# baseline (speedup 1.0000x reference)
import jax
import jax.numpy as jnp
from jax import lax
from jax.experimental import pallas as pl
from jax.experimental.pallas import tpu as pltpu
from jax.experimental.pallas import tpu_sc as plsc

NSEL = 300
NB = 16
NFLAT = 91000
NPAD = 91008
NVREG = NPAD // 16
NCLS = 91
CAP = 512
CBUF = 544
OPAD = 304
IMIN = -2147483648
K05 = 0x3F000000
NB0 = 544
HISTW = NB0 * 16


def _mono(b):
    return b ^ (lax.shift_right_arithmetic(b, 31) & jnp.int32(0x7FFFFFFF))


def _sc_body(prob_hbm, boxes_hbm, ts_hbm,
             out_s_hbm, out_l_hbm, out_b_hbm,
             data, boxv, tsv, hist, totals,
             ckey, cidx, crank, stage_s, stage_l, stage_b, st):
    wid = lax.axis_index("s") * 2 + lax.axis_index("c")

    lanes = lax.broadcasted_iota(jnp.int32, (16,), 0)
    zeros = jnp.zeros((16,), jnp.int32)
    ones = jnp.full((16,), 1, jnp.int32)

    @pl.when(wid < NB)
    def _worker():
        b = wid

        pltpu.sync_copy(ts_hbm, tsv.at[pl.ds(0, 32)])
        pltpu.sync_copy(boxes_hbm.at[pl.ds(b * 4000, 4000)], boxv)
        pltpu.sync_copy(prob_hbm.at[pl.ds(b * NFLAT, NFLAT)],
                        data.at[pl.ds(0, NFLAT)])
        plsc.store_scatter(data, [jnp.full((16,), NFLAT - 8, jnp.int32) + lanes],
                           jnp.full((16,), -1, jnp.int32), mask=lanes >= 8)

        def zero_hist(nreg):
            def zh(j, c):
                hist[pl.ds(j * 16, 16)] = zeros
                return c
            lax.fori_loop(0, nreg, zh, 0)

        zero_hist(NB0)

        def p_a(i, c):
            for u in range(8):
                o = (i * 8 + u) * 16
                key = _mono(data[pl.ds(o, 16)])
                d = lax.shift_right_arithmetic(jnp.maximum(key, K05) - K05, 14)
                d = jnp.minimum(d, NB0 - 1)
                plsc.addupdate_scatter(hist, [lax.shift_left(d, 4) | lanes], ones)
            return c
        lax.fori_loop(0, NVREG // 8, p_a, 0)
        zero_hist(NB0)
        lax.fori_loop(0, NVREG // 8, p_a, 0)

        def scan_hist(need, nreg):
            def tot(j, c):
                base = (j * 16 + lanes) * 16
                acc = plsc.load_gather(hist, [base])
                for l in range(1, 16):
                    acc = acc + plsc.load_gather(hist, [base + l])
                totals[pl.ds(j * 16, 16)] = acc
                return c
            lax.fori_loop(0, nreg, tot, 0)

            def scn(i, carry):
                q, a, e, c = carry
                j = nreg - 1 - i
                v = totals[pl.ds(j * 16, 16)]
                incl = lax.rev(plsc.cumsum(lax.rev(v, (0,))), (0,)) + c
                excl = incl - v
                m = (excl < need) & (incl >= need)
                has = jnp.max(jnp.where(m, 1, 0))
                qq = j * 16 + jnp.max(jnp.where(m, lanes, 0))
                aa = jnp.max(jnp.where(m, excl, 0))
                ee = jnp.max(jnp.where(m, v, 0))
                q = jnp.where(has > 0, qq, q)
                a = jnp.where(has > 0, aa, a)
                e = jnp.where(has > 0, ee, e)
                return q, a, e, c + jnp.sum(v)
            q, a, e, _ = lax.fori_loop(0, nreg, scn, (0, 0, 0, 0))
            return q, a, e

        q0, a0, e0 = scan_hist(jnp.int32(NSEL), NB0 // 16)
        ok = (q0 > 0) & (a0 + e0 <= CAP)
        st[3] = jnp.where(ok, 1, 0)
        st[4] = K05 + lax.shift_left(q0, 14)
        st[5] = 0

        def level(shift, first, is_last):
            @pl.when(st[3] == 0)
            def _():
                if first:
                    prefix = jnp.int32(0)
                    need = jnp.int32(NSEL)
                    sure = jnp.int32(0)
                else:
                    prefix = st[0]
                    need = st[1]
                    sure = st[2]
                zero_hist(16)

                def p_b(i, c):
                    for u in range(4):
                        key = _mono(data[pl.ds((i * 4 + u) * 16, 16)])
                        if first:
                            d = lax.shift_right_arithmetic(key, 24) + 128
                            plsc.addupdate_scatter(
                                hist, [lax.shift_left(d, 4) | lanes], ones)
                        else:
                            m = lax.shift_right_arithmetic(
                                key, shift + 8) == prefix
                            d = lax.shift_right_arithmetic(key, shift) & 255
                            plsc.addupdate_scatter(
                                hist, [lax.shift_left(d, 4) | lanes], ones,
                                mask=m)
                    return c
                lax.fori_loop(0, NVREG // 4, p_b, 0)

                q, a, e = scan_hist(need, 16)
                if first:
                    pref2 = q - 128
                else:
                    pref2 = lax.shift_left(prefix, 8) | q
                st[0] = pref2
                st[1] = need - a
                st[2] = sure + a
                cc = sure + a + e
                st[3] = 1 if is_last else jnp.where(cc <= CAP, 1, 0)
                st[5] = jnp.where(cc > CAP, 1, 0) if is_last else 0
                st[4] = lax.shift_left(pref2, shift)

        level(24, True, False)
        level(16, False, False)
        level(8, False, False)
        level(0, False, True)

        thr = st[4]
        exact = st[5]
        need_eq = st[1]

        @pl.when(exact == 0)
        def _():
            def c_a(i, off):
                ks = [_mono(data[pl.ds((i * 4 + u) * 16, 16)]) for u in range(4)]
                sels = [k >= thr for k in ks]
                pcs = [plsc.all_reduce_population_count(s)[0] for s in sels]
                o = off
                for u in range(4):
                    plsc.store_compressed(ckey.at[pl.ds(o, 16)], ks[u],
                                          mask=sels[u])
                    plsc.store_compressed(cidx.at[pl.ds(o, 16)],
                                          (i * 4 + u) * 16 + lanes,
                                          mask=sels[u])
                    o = o + pcs[u]
                return o
            st[6] = lax.fori_loop(0, NVREG // 4, c_a, 0)

        @pl.when(exact == 1)
        def _():
            def c_b(i, carry):
                off, taken = carry
                for u in range(4):
                    ii = i * 4 + u
                    key = _mono(data[pl.ds(ii * 16, 16)])
                    gt = key > thr
                    eq = key == thr
                    pceq = plsc.cumsum(jnp.where(eq, 1, 0))
                    seleq = eq & ((taken + pceq) <= need_eq)
                    sel = gt | seleq
                    selv = jnp.where(sel, 1, 0)
                    pos = off + plsc.cumsum(selv) - selv
                    plsc.store_scatter(ckey, [pos], key, mask=sel)
                    plsc.store_scatter(cidx, [pos], ii * 16 + lanes, mask=sel)
                    off = off + plsc.all_reduce_population_count(sel)
                    taken = taken + plsc.all_reduce_population_count(seleq)
                return off, taken
            offv, _t = lax.fori_loop(0, NVREG // 4, c_b, (zeros, zeros))
            st[6] = offv[0]

        nc = st[6]
        plsc.store_scatter(ckey, [nc + lanes], jnp.full((16,), IMIN, jnp.int32))
        plsc.store_scatter(cidx, [nc + lanes], zeros)

        nv = (nc + 15) // 16

        def rk_outer(je, c):
            ke = ckey[pl.ds(je * 16, 16)]
            pose = je * 16 + lanes

            def rk_inner(jf, acc):
                base = jf * 16
                for r in range(16):
                    idx = base + ((lanes + r) & 15)
                    kv = plsc.load_gather(ckey, [idx])
                    hit = (kv > ke) | ((kv == ke) & (idx < pose))
                    acc = acc + jnp.where(hit, 1, 0)
                return acc
            acc = lax.fori_loop(0, nv, rk_inner, zeros)
            crank[pl.ds(je * 16, 16)] = acc
            return c
        lax.fori_loop(0, nv, rk_outer, 0)

        tsx = tsv[pl.ds(2 * b, 16)]
        hf = tsx[0].astype(jnp.float32)
        wf = tsx[1].astype(jnp.float32)
        recip = jnp.float32(1.0 / NCLS)

        def post(j, c):
            key = ckey[pl.ds(j * 16, 16)]
            idxv = cidx[pl.ds(j * 16, 16)]
            rk = crank[pl.ds(j * 16, 16)]
            posv = j * 16 + lanes
            m = (rk < NSEL) & (posv < nc)
            score = plsc.bitcast(_mono(key), jnp.float32)
            qf = (idxv.astype(jnp.float32) * recip).astype(jnp.int32)
            r = idxv - qf * NCLS
            qq = qf + jnp.where(r >= NCLS, 1, 0) - jnp.where(r < 0, 1, 0)
            cls = idxv - qq * NCLS
            a0_ = lax.shift_left(qq, 2)
            cx = plsc.load_gather(boxv, [a0_], mask=m)
            cy = plsc.load_gather(boxv, [a0_ + 1], mask=m)
            w_ = plsc.load_gather(boxv, [a0_ + 2], mask=m)
            h_ = plsc.load_gather(boxv, [a0_ + 3], mask=m)
            x1 = (cx - 0.5 * w_) * wf
            y1 = (cy - 0.5 * h_) * hf
            x2 = (cx + 0.5 * w_) * wf
            y2 = (cy + 0.5 * h_) * hf
            plsc.store_scatter(stage_s, [rk], score, mask=m)
            plsc.store_scatter(stage_l, [rk], cls, mask=m)
            rb = lax.shift_left(rk, 2)
            plsc.store_scatter(stage_b, [rb], x1, mask=m)
            plsc.store_scatter(stage_b, [rb + 1], y1, mask=m)
            plsc.store_scatter(stage_b, [rb + 2], x2, mask=m)
            plsc.store_scatter(stage_b, [rb + 3], y2, mask=m)
            return c
        lax.fori_loop(0, nv, post, 0)

        pltpu.sync_copy(stage_s, out_s_hbm.at[pl.ds(b * OPAD, OPAD)])
        pltpu.sync_copy(stage_l, out_l_hbm.at[pl.ds(b * OPAD, OPAD)])
        pltpu.sync_copy(stage_b, out_b_hbm.at[pl.ds(b * OPAD * 4, OPAD * 4)])


@jax.jit
def _sc_topk(prob_bits, boxes_flat, ts_flat):
    mesh = plsc.VectorSubcoreMesh(core_axis_name="c", subcore_axis_name="s",
                                  num_cores=2, num_subcores=16)
    fn = pl.kernel(
        _sc_body,
        out_type=(
            jax.ShapeDtypeStruct((NB * OPAD,), jnp.float32),
            jax.ShapeDtypeStruct((NB * OPAD,), jnp.int32),
            jax.ShapeDtypeStruct((NB * OPAD * 4,), jnp.float32),
        ),
        mesh=mesh,
        compiler_params=pltpu.CompilerParams(needs_layout_passes=False),
        scratch_types=[
            pltpu.VMEM((NPAD,), jnp.int32),
            pltpu.VMEM((4000,), jnp.float32),
            pltpu.VMEM((48,), jnp.int32),
            pltpu.VMEM((HISTW,), jnp.int32),
            pltpu.VMEM((NB0,), jnp.int32),
            pltpu.VMEM((CBUF,), jnp.int32),
            pltpu.VMEM((CBUF,), jnp.int32),
            pltpu.VMEM((CBUF,), jnp.int32),
            pltpu.VMEM((OPAD,), jnp.float32),
            pltpu.VMEM((OPAD,), jnp.int32),
            pltpu.VMEM((OPAD * 4,), jnp.float32),
            pltpu.SMEM((8,), jnp.int32),
        ],
    )
    return fn(prob_bits, boxes_flat, ts_flat)


def kernel(pred_logits, pred_boxes, target_sizes):
    B, N, C = pred_logits.shape
    prob = jax.nn.sigmoid(pred_logits)
    prob_bits = lax.bitcast_convert_type(prob, jnp.int32).reshape(B * N * C)
    scores_p, labels_p, boxes_p = _sc_topk(
        prob_bits, pred_boxes.reshape(-1), target_sizes.reshape(-1))
    scores = scores_p.reshape(NB, OPAD)[:, :NSEL]
    labels = labels_p.reshape(NB, OPAD)[:, :NSEL]
    boxes = boxes_p.reshape(NB, OPAD, 4)[:, :NSEL, :]
    return scores, labels, boxes


# speedup vs baseline: 2.3453x; 2.3453x over previous
import jax
import jax.numpy as jnp
from jax import lax
from jax.experimental import pallas as pl
from jax.experimental.pallas import tpu as pltpu
from jax.experimental.pallas import tpu_sc as plsc

NSEL = 300
NB = 16
NFLAT = 91000
NPAD = 91008
NVREG = NPAD // 16
NCLS = 91
CAP = 512
CBUF = 544
OPAD = 304
IMIN = -2147483648
K05 = 0x3F000000
NB0 = 544
HISTW = NB0 * 16


def _sc_body(prob_hbm, boxes_hbm, ts_hbm,
             out_s_hbm, out_l_hbm, out_b_hbm,
             data, boxv, tsv, hist, totals,
             ckey, cidx, crank, stage_s, stage_l, stage_b, st):
    wid = lax.axis_index("s") * 2 + lax.axis_index("c")

    lanes = lax.broadcasted_iota(jnp.int32, (16,), 0)
    zeros = jnp.zeros((16,), jnp.int32)
    ones = jnp.full((16,), 1, jnp.int32)

    @pl.when(wid < NB)
    def _worker():
        b = wid

        pltpu.sync_copy(ts_hbm, tsv.at[pl.ds(0, 32)])
        pltpu.sync_copy(boxes_hbm.at[pl.ds(b * 4000, 4000)], boxv)
        pltpu.sync_copy(prob_hbm.at[pl.ds(b * NFLAT, NFLAT)],
                        data.at[pl.ds(0, NFLAT)])
        plsc.store_scatter(data, [jnp.full((16,), NFLAT - 8, jnp.int32) + lanes],
                           jnp.full((16,), -1, jnp.int32), mask=lanes >= 8)

        def zero_hist(nreg):
            @plsc.parallel_loop(0, nreg, unroll=8)
            def _zh(j):
                hist[pl.ds(j * 16, 16)] = zeros

        zero_hist(NB0)

        @plsc.parallel_loop(0, NVREG, unroll=8)
        def _pa(i):
            key = data[pl.ds(i * 16, 16)]
            d = lax.shift_right_logical(jnp.maximum(key, K05) - K05, 14)
            d = jnp.minimum(d, NB0 - 1)
            plsc.addupdate_scatter(hist, [lax.shift_left(d, 4) | lanes], ones)

        def scan_hist(need, nreg):
            @plsc.parallel_loop(0, nreg, unroll=2)
            def _tot(j):
                base = (j * 16 + lanes) * 16
                acc = plsc.load_gather(hist, [base])
                for l in range(1, 16):
                    acc = acc + plsc.load_gather(hist, [base + l])
                totals[pl.ds(j * 16, 16)] = acc

            def scn(i, carry):
                q, a, e, c = carry
                j = nreg - 1 - i
                v = totals[pl.ds(j * 16, 16)]
                incl = lax.rev(plsc.cumsum(lax.rev(v, (0,))), (0,)) + c
                excl = incl - v
                m = (excl < need) & (incl >= need)
                has = jnp.max(jnp.where(m, 1, 0))
                qq = j * 16 + jnp.max(jnp.where(m, lanes, 0))
                aa = jnp.max(jnp.where(m, excl, 0))
                ee = jnp.max(jnp.where(m, v, 0))
                q = jnp.where(has > 0, qq, q)
                a = jnp.where(has > 0, aa, a)
                e = jnp.where(has > 0, ee, e)
                return q, a, e, c + jnp.sum(v)
            q, a, e, _ = lax.fori_loop(0, nreg, scn, (0, 0, 0, 0))
            return q, a, e

        q0, a0, e0 = scan_hist(jnp.int32(NSEL), NB0 // 16)
        ok = (q0 > 0) & (a0 + e0 <= CAP)
        st[3] = jnp.where(ok, 1, 0)
        st[4] = K05 + lax.shift_left(q0, 14)
        st[5] = 0

        def level(shift, first, is_last):
            @pl.when(st[3] == 0)
            def _():
                if first:
                    prefix = jnp.int32(0)
                    need = jnp.int32(NSEL)
                    sure = jnp.int32(0)
                else:
                    prefix = st[0]
                    need = st[1]
                    sure = st[2]
                zero_hist(16)

                @plsc.parallel_loop(0, NVREG, unroll=8)
                def _pb(i):
                    key = data[pl.ds(i * 16, 16)]
                    if first:
                        d = lax.shift_right_arithmetic(key, 24) + 128
                        plsc.addupdate_scatter(
                            hist, [lax.shift_left(d, 4) | lanes], ones)
                    else:
                        m = lax.shift_right_arithmetic(key, shift + 8) == prefix
                        d = lax.shift_right_arithmetic(key, shift) & 255
                        plsc.addupdate_scatter(
                            hist, [lax.shift_left(d, 4) | lanes], ones, mask=m)

                q, a, e = scan_hist(need, 16)
                if first:
                    pref2 = q - 128
                else:
                    pref2 = lax.shift_left(prefix, 8) | q
                st[0] = pref2
                st[1] = need - a
                st[2] = sure + a
                cc = sure + a + e
                st[3] = 1 if is_last else jnp.where(cc <= CAP, 1, 0)
                st[5] = jnp.where(cc > CAP, 1, 0) if is_last else 0
                st[4] = lax.shift_left(pref2, shift)

        level(24, True, False)
        level(16, False, False)
        level(8, False, False)
        level(0, False, True)

        thr = st[4]
        exact = st[5]
        need_eq = st[1]

        @pl.when(exact == 0)
        def _():
            @plsc.parallel_loop(0, NVREG // 4, carry=jnp.int32(0))
            def c_a(i, off):
                ks = [data[pl.ds((i * 4 + u) * 16, 16)] for u in range(4)]
                sels = [k >= thr for k in ks]
                pcs = [plsc.all_reduce_population_count(s)[0] for s in sels]
                o = off
                for u in range(4):
                    plsc.store_compressed(ckey.at[pl.ds(o, 16)], ks[u],
                                          mask=sels[u])
                    plsc.store_compressed(cidx.at[pl.ds(o, 16)],
                                          (i * 4 + u) * 16 + lanes,
                                          mask=sels[u])
                    o = o + pcs[u]
                return o
            st[6] = c_a

        @pl.when(exact == 1)
        def _():
            def c_b(i, carry):
                off, taken = carry
                for u in range(4):
                    ii = i * 4 + u
                    key = data[pl.ds(ii * 16, 16)]
                    gt = key > thr
                    eq = key == thr
                    pceq = plsc.cumsum(jnp.where(eq, 1, 0))
                    seleq = eq & ((taken + pceq) <= need_eq)
                    sel = gt | seleq
                    plsc.store_compressed(ckey.at[pl.ds(off, 16)], key,
                                          mask=sel)
                    plsc.store_compressed(cidx.at[pl.ds(off, 16)],
                                          ii * 16 + lanes, mask=sel)
                    off = off + plsc.all_reduce_population_count(sel)[0]
                    taken = taken + plsc.all_reduce_population_count(seleq)[0]
                return off, taken
            cc, _t = lax.fori_loop(0, NVREG // 4, c_b, (0, 0))
            st[6] = cc

        nc = st[6]
        plsc.store_scatter(ckey, [nc + lanes], jnp.full((16,), IMIN, jnp.int32))
        plsc.store_scatter(cidx, [nc + lanes], zeros)

        nv = (nc + 15) // 16

        @plsc.parallel_loop(0, nv)
        def _rk(je):
            ke = ckey[pl.ds(je * 16, 16)]
            pose = je * 16 + lanes

            @plsc.parallel_loop(0, nv, carry=zeros)
            def rk_inner(jf, acc):
                base = jf * 16
                for r in range(16):
                    idx = base + ((lanes + r) & 15)
                    kv = plsc.load_gather(ckey, [idx])
                    hit = (kv > ke) | ((kv == ke) & (idx < pose))
                    acc = acc + jnp.where(hit, 1, 0)
                return acc
            crank[pl.ds(je * 16, 16)] = rk_inner

        tsx = tsv[pl.ds(2 * b, 16)]
        hf = tsx[0].astype(jnp.float32)
        wf = tsx[1].astype(jnp.float32)
        recip = jnp.float32(1.0 / NCLS)

        @plsc.parallel_loop(0, nv, unroll=2)
        def _post(j):
            key = ckey[pl.ds(j * 16, 16)]
            idxv = cidx[pl.ds(j * 16, 16)]
            rk = crank[pl.ds(j * 16, 16)]
            posv = j * 16 + lanes
            m = (rk < NSEL) & (posv < nc)
            score = plsc.bitcast(key, jnp.float32)
            qf = (idxv.astype(jnp.float32) * recip).astype(jnp.int32)
            r = idxv - qf * NCLS
            qq = qf + jnp.where(r >= NCLS, 1, 0) - jnp.where(r < 0, 1, 0)
            cls = idxv - qq * NCLS
            a0_ = lax.shift_left(qq, 2)
            cx = plsc.load_gather(boxv, [a0_], mask=m)
            cy = plsc.load_gather(boxv, [a0_ + 1], mask=m)
            w_ = plsc.load_gather(boxv, [a0_ + 2], mask=m)
            h_ = plsc.load_gather(boxv, [a0_ + 3], mask=m)
            x1 = (cx - 0.5 * w_) * wf
            y1 = (cy - 0.5 * h_) * hf
            x2 = (cx + 0.5 * w_) * wf
            y2 = (cy + 0.5 * h_) * hf
            plsc.store_scatter(stage_s, [rk], score, mask=m)
            plsc.store_scatter(stage_l, [rk], cls, mask=m)
            rb = lax.shift_left(rk, 2)
            plsc.store_scatter(stage_b, [rb], x1, mask=m)
            plsc.store_scatter(stage_b, [rb + 1], y1, mask=m)
            plsc.store_scatter(stage_b, [rb + 2], x2, mask=m)
            plsc.store_scatter(stage_b, [rb + 3], y2, mask=m)

        pltpu.sync_copy(stage_s, out_s_hbm.at[pl.ds(b * OPAD, OPAD)])
        pltpu.sync_copy(stage_l, out_l_hbm.at[pl.ds(b * OPAD, OPAD)])
        pltpu.sync_copy(stage_b, out_b_hbm.at[pl.ds(b * OPAD * 4, OPAD * 4)])


@jax.jit
def _sc_topk(prob_bits, boxes_flat, ts_flat):
    mesh = plsc.VectorSubcoreMesh(core_axis_name="c", subcore_axis_name="s",
                                  num_cores=2, num_subcores=16)
    fn = pl.kernel(
        _sc_body,
        out_type=(
            jax.ShapeDtypeStruct((NB * OPAD,), jnp.float32),
            jax.ShapeDtypeStruct((NB * OPAD,), jnp.int32),
            jax.ShapeDtypeStruct((NB * OPAD * 4,), jnp.float32),
        ),
        mesh=mesh,
        compiler_params=pltpu.CompilerParams(needs_layout_passes=False),
        scratch_types=[
            pltpu.VMEM((NPAD,), jnp.int32),
            pltpu.VMEM((4000,), jnp.float32),
            pltpu.VMEM((48,), jnp.int32),
            pltpu.VMEM((HISTW,), jnp.int32),
            pltpu.VMEM((NB0,), jnp.int32),
            pltpu.VMEM((CBUF,), jnp.int32),
            pltpu.VMEM((CBUF,), jnp.int32),
            pltpu.VMEM((CBUF,), jnp.int32),
            pltpu.VMEM((OPAD,), jnp.float32),
            pltpu.VMEM((OPAD,), jnp.int32),
            pltpu.VMEM((OPAD * 4,), jnp.float32),
            pltpu.SMEM((8,), jnp.int32),
        ],
    )
    return fn(prob_bits, boxes_flat, ts_flat)


def kernel(pred_logits, pred_boxes, target_sizes):
    B, N, C = pred_logits.shape
    prob = jax.nn.sigmoid(pred_logits)
    prob_bits = lax.bitcast_convert_type(prob, jnp.int32).reshape(B * N * C)
    scores_p, labels_p, boxes_p = _sc_topk(
        prob_bits, pred_boxes.reshape(-1), target_sizes.reshape(-1))
    scores = scores_p.reshape(NB, OPAD)[:, :NSEL]
    labels = labels_p.reshape(NB, OPAD)[:, :NSEL]
    boxes = boxes_p.reshape(NB, OPAD, 4)[:, :NSEL, :]
    return scores, labels, boxes


# speedup vs baseline: 2.3468x; 1.0006x over previous
import jax
import jax.numpy as jnp
from jax import lax
from jax.experimental import pallas as pl
from jax.experimental.pallas import tpu as pltpu
from jax.experimental.pallas import tpu_sc as plsc

NSEL = 300
NB = 16
NFLAT = 91000
NPAD = 91008
NVREG = NPAD // 16
NCLS = 91
CAP = 512
CBUF = 544
OPAD = 304
IMIN = -2147483648
K05 = 0x3F000000
NB0 = 544
HISTW = NB0 * 16


def _sc_body(prob_hbm, boxes_hbm, ts_hbm,
             out_s_hbm, out_l_hbm, out_b_hbm,
             data, boxv, tsv, hist, totals,
             ckey, cidx, crank, stage_s, stage_l, stage_b, st):
    wid = lax.axis_index("s") * 2 + lax.axis_index("c")

    lanes = lax.broadcasted_iota(jnp.int32, (16,), 0)
    zeros = jnp.zeros((16,), jnp.int32)
    ones = jnp.full((16,), 1, jnp.int32)

    @pl.when(wid < NB)
    def _worker():
        b = wid

        pltpu.sync_copy(ts_hbm, tsv.at[pl.ds(0, 32)])
        pltpu.sync_copy(boxes_hbm.at[pl.ds(b * 4000, 4000)], boxv)
        pltpu.sync_copy(prob_hbm.at[pl.ds(b * NFLAT, NFLAT)],
                        data.at[pl.ds(0, NFLAT)])
        plsc.store_scatter(data, [jnp.full((16,), NFLAT - 8, jnp.int32) + lanes],
                           jnp.full((16,), -1, jnp.int32), mask=lanes >= 8)

        def zero_hist(nreg):
            @plsc.parallel_loop(0, nreg, unroll=8)
            def _zh(j):
                hist[pl.ds(j * 16, 16)] = zeros

        zero_hist(NB0)

        @plsc.parallel_loop(0, NVREG, unroll=8)
        def _pa(i):
            key = data[pl.ds(i * 16, 16)]
            d = lax.shift_right_logical(jnp.maximum(key, K05) - K05, 14)
            d = jnp.minimum(d, NB0 - 1)
            plsc.addupdate_scatter(hist, [lax.shift_left(d, 4) | lanes], ones)

        def scan_hist(need, nreg):
            @plsc.parallel_loop(0, nreg, unroll=2)
            def _tot(j):
                base = (j * 16 + lanes) * 16
                acc = plsc.load_gather(hist, [base])
                for l in range(1, 16):
                    acc = acc + plsc.load_gather(hist, [base + l])
                totals[pl.ds(j * 16, 16)] = acc

            def scn(i, carry):
                q, a, e, c = carry
                j = nreg - 1 - i
                v = totals[pl.ds(j * 16, 16)]
                incl = lax.rev(plsc.cumsum(lax.rev(v, (0,))), (0,)) + c
                excl = incl - v
                m = (excl < need) & (incl >= need)
                has = jnp.max(jnp.where(m, 1, 0))
                qq = j * 16 + jnp.max(jnp.where(m, lanes, 0))
                aa = jnp.max(jnp.where(m, excl, 0))
                ee = jnp.max(jnp.where(m, v, 0))
                q = jnp.where(has > 0, qq, q)
                a = jnp.where(has > 0, aa, a)
                e = jnp.where(has > 0, ee, e)
                return q, a, e, c + jnp.sum(v)
            q, a, e, _ = lax.fori_loop(0, nreg, scn, (0, 0, 0, 0))
            return q, a, e

        q0, a0, e0 = scan_hist(jnp.int32(NSEL), NB0 // 16)
        ok = (q0 > 0) & (a0 + e0 <= CAP)
        st[3] = jnp.where(ok, 1, 0)
        st[4] = K05 + lax.shift_left(q0, 14)
        st[5] = 0

        def level(shift, first, is_last):
            @pl.when(st[3] == 0)
            def _():
                if first:
                    prefix = jnp.int32(0)
                    need = jnp.int32(NSEL)
                    sure = jnp.int32(0)
                else:
                    prefix = st[0]
                    need = st[1]
                    sure = st[2]
                zero_hist(16)

                @plsc.parallel_loop(0, NVREG, unroll=8)
                def _pb(i):
                    key = data[pl.ds(i * 16, 16)]
                    if first:
                        d = lax.shift_right_arithmetic(key, 24) + 128
                        plsc.addupdate_scatter(
                            hist, [lax.shift_left(d, 4) | lanes], ones)
                    else:
                        m = lax.shift_right_arithmetic(key, shift + 8) == prefix
                        d = lax.shift_right_arithmetic(key, shift) & 255
                        plsc.addupdate_scatter(
                            hist, [lax.shift_left(d, 4) | lanes], ones, mask=m)

                q, a, e = scan_hist(need, 16)
                if first:
                    pref2 = q - 128
                else:
                    pref2 = lax.shift_left(prefix, 8) | q
                st[0] = pref2
                st[1] = need - a
                st[2] = sure + a
                cc = sure + a + e
                st[3] = 1 if is_last else jnp.where(cc <= CAP, 1, 0)
                st[5] = jnp.where(cc > CAP, 1, 0) if is_last else 0
                st[4] = lax.shift_left(pref2, shift)

        level(24, True, False)
        level(16, False, False)
        level(8, False, False)
        level(0, False, True)

        thr = st[4]
        exact = st[5]
        need_eq = st[1]

        @pl.when(exact == 0)
        def _():
            @plsc.parallel_loop(0, NVREG // 4, carry=jnp.int32(0))
            def c_a(i, off):
                ks = [data[pl.ds((i * 4 + u) * 16, 16)] for u in range(4)]
                sels = [k >= thr for k in ks]
                pcs = [plsc.all_reduce_population_count(s)[0] for s in sels]
                o = off
                for u in range(4):
                    plsc.store_compressed(ckey.at[pl.ds(o, 16)], ks[u],
                                          mask=sels[u])
                    plsc.store_compressed(cidx.at[pl.ds(o, 16)],
                                          (i * 4 + u) * 16 + lanes,
                                          mask=sels[u])
                    o = o + pcs[u]
                return o
            st[6] = c_a

        @pl.when(exact == 1)
        def _():
            def c_b(i, carry):
                off, taken = carry
                for u in range(4):
                    ii = i * 4 + u
                    key = data[pl.ds(ii * 16, 16)]
                    gt = key > thr
                    eq = key == thr
                    pceq = plsc.cumsum(jnp.where(eq, 1, 0))
                    seleq = eq & ((taken + pceq) <= need_eq)
                    sel = gt | seleq
                    plsc.store_compressed(ckey.at[pl.ds(off, 16)], key,
                                          mask=sel)
                    plsc.store_compressed(cidx.at[pl.ds(off, 16)],
                                          ii * 16 + lanes, mask=sel)
                    off = off + plsc.all_reduce_population_count(sel)[0]
                    taken = taken + plsc.all_reduce_population_count(seleq)[0]
                return off, taken
            cc, _t = lax.fori_loop(0, NVREG // 4, c_b, (0, 0))
            st[6] = cc

        nc = st[6]
        plsc.store_scatter(ckey, [nc + lanes], jnp.full((16,), IMIN, jnp.int32))
        plsc.store_scatter(cidx, [nc + lanes], zeros)

        nv = (nc + 15) // 16

        @plsc.parallel_loop(0, nv)
        def _rk(je):
            ke = ckey[pl.ds(je * 16, 16)]
            pose = je * 16 + lanes

            @plsc.parallel_loop(0, nv, carry=zeros)
            def rk_inner(jf, acc):
                base = jf * 16
                for r in range(16):
                    idx = base + ((lanes + r) & 15)
                    kv = plsc.load_gather(ckey, [idx])
                    hit = (kv > ke) | ((kv == ke) & (idx < pose))
                    acc = acc + jnp.where(hit, 1, 0)
                return acc
            crank[pl.ds(je * 16, 16)] = rk_inner

        tsx = tsv[pl.ds(2 * b, 16)]
        hf = tsx[0].astype(jnp.float32)
        wf = tsx[1].astype(jnp.float32)
        recip = jnp.float32(1.0 / NCLS)

        @plsc.parallel_loop(0, nv, unroll=2)
        def _post(j):
            key = ckey[pl.ds(j * 16, 16)]
            idxv = cidx[pl.ds(j * 16, 16)]
            rk = crank[pl.ds(j * 16, 16)]
            posv = j * 16 + lanes
            m = (rk < NSEL) & (posv < nc)
            score = plsc.bitcast(key, jnp.float32)
            qf = (idxv.astype(jnp.float32) * recip).astype(jnp.int32)
            r = idxv - qf * NCLS
            qq = qf + jnp.where(r >= NCLS, 1, 0) - jnp.where(r < 0, 1, 0)
            cls = idxv - qq * NCLS
            a0_ = lax.shift_left(qq, 2)
            cx = plsc.load_gather(boxv, [a0_], mask=m)
            cy = plsc.load_gather(boxv, [a0_ + 1], mask=m)
            w_ = plsc.load_gather(boxv, [a0_ + 2], mask=m)
            h_ = plsc.load_gather(boxv, [a0_ + 3], mask=m)
            x1 = (cx - 0.5 * w_) * wf
            y1 = (cy - 0.5 * h_) * hf
            x2 = (cx + 0.5 * w_) * wf
            y2 = (cy + 0.5 * h_) * hf
            plsc.store_scatter(stage_s, [rk], score, mask=m)
            plsc.store_scatter(stage_l, [rk], cls, mask=m)
            rb = lax.shift_left(rk, 2)
            plsc.store_scatter(stage_b, [rb], x1, mask=m)
            plsc.store_scatter(stage_b, [rb + 1], y1, mask=m)
            plsc.store_scatter(stage_b, [rb + 2], x2, mask=m)
            plsc.store_scatter(stage_b, [rb + 3], y2, mask=m)

        pltpu.sync_copy(stage_s, out_s_hbm.at[pl.ds(b * OPAD, OPAD)])
        pltpu.sync_copy(stage_l, out_l_hbm.at[pl.ds(b * OPAD, OPAD)])
        pltpu.sync_copy(stage_b, out_b_hbm.at[pl.ds(b * OPAD * 4, OPAD * 4)])


@jax.jit
def _sc_topk(prob_bits, boxes_flat, ts_flat):
    mesh = plsc.VectorSubcoreMesh(core_axis_name="c", subcore_axis_name="s",
                                  num_cores=2, num_subcores=16)
    fn = pl.kernel(
        _sc_body,
        out_type=(
            jax.ShapeDtypeStruct((NB * OPAD,), jnp.float32),
            jax.ShapeDtypeStruct((NB * OPAD,), jnp.int32),
            jax.ShapeDtypeStruct((NB * OPAD * 4,), jnp.float32),
        ),
        mesh=mesh,
        compiler_params=pltpu.CompilerParams(needs_layout_passes=False),
        scratch_types=[
            pltpu.VMEM((NPAD,), jnp.int32),
            pltpu.VMEM((4000,), jnp.float32),
            pltpu.VMEM((48,), jnp.int32),
            pltpu.VMEM((HISTW,), jnp.int32),
            pltpu.VMEM((NB0,), jnp.int32),
            pltpu.VMEM((CBUF,), jnp.int32),
            pltpu.VMEM((CBUF,), jnp.int32),
            pltpu.VMEM((CBUF,), jnp.int32),
            pltpu.VMEM((OPAD,), jnp.float32),
            pltpu.VMEM((OPAD,), jnp.int32),
            pltpu.VMEM((OPAD * 4,), jnp.float32),
            pltpu.SMEM((8,), jnp.int32),
        ],
    )
    return fn(prob_bits, boxes_flat, ts_flat)


def kernel(pred_logits, pred_boxes, target_sizes):
    B, N, C = pred_logits.shape
    prob = jax.nn.sigmoid(pred_logits.reshape(B * N * C))
    prob_bits = lax.bitcast_convert_type(prob, jnp.int32)
    scores_p, labels_p, boxes_p = _sc_topk(
        prob_bits, pred_boxes.reshape(-1), target_sizes.reshape(-1))
    scores = scores_p.reshape(NB, OPAD)[:, :NSEL]
    labels = labels_p.reshape(NB, OPAD)[:, :NSEL]
    boxes = boxes_p.reshape(NB, OPAD, 4)[:, :NSEL, :]
    return scores, labels, boxes


# speedup vs baseline: 2.3850x; 1.0163x over previous
import jax
import jax.numpy as jnp
from jax import lax
from jax.experimental import pallas as pl
from jax.experimental.pallas import tpu as pltpu
from jax.experimental.pallas import tpu_sc as plsc

NSEL = 300
NB = 16
NFLAT = 91000
NPAD = 91008
NVREG = NPAD // 16
NCLS = 91
CAP = 512
CBUF = 544
OPAD = 304
IMIN = -2147483648
K05 = 0x3F000000
NB0 = 544
HISTW = NB0 * 16


def _sc_body(prob_hbm, boxes_hbm, ts_hbm,
             out_s_hbm, out_l_hbm, out_b_hbm,
             data, boxv, tsv, hist, totals,
             ckey, cidx, crank, stage_s, stage_l, stage_b, st):
    wid = lax.axis_index("s") * 2 + lax.axis_index("c")

    lanes = lax.broadcasted_iota(jnp.int32, (16,), 0)
    zeros = jnp.zeros((16,), jnp.int32)
    ones = jnp.full((16,), 1, jnp.int32)

    @pl.when(wid < NB)
    def _worker():
        b = wid

        pltpu.sync_copy(ts_hbm, tsv.at[pl.ds(0, 32)])
        pltpu.sync_copy(boxes_hbm.at[pl.ds(b * 4000, 4000)], boxv)
        pltpu.sync_copy(prob_hbm.at[pl.ds(b * NFLAT, NFLAT)],
                        data.at[pl.ds(0, NFLAT)])
        plsc.store_scatter(data, [jnp.full((16,), NFLAT - 8, jnp.int32) + lanes],
                           jnp.full((16,), -1, jnp.int32), mask=lanes >= 8)

        def zero_hist(nreg):
            @plsc.parallel_loop(0, nreg, unroll=8)
            def _zh(j):
                hist[pl.ds(j * 16, 16)] = zeros

        zero_hist(NB0)

        @plsc.parallel_loop(0, NVREG, unroll=8)
        def _pa(i):
            key = data[pl.ds(i * 16, 16)]
            d = lax.shift_right_logical(jnp.maximum(key, K05) - K05, 14)
            d = jnp.minimum(d, NB0 - 1)
            plsc.addupdate_scatter(hist, [lax.shift_left(d, 4) | lanes], ones)

        def scan_hist(need, nreg):
            @plsc.parallel_loop(0, nreg, unroll=2)
            def _tot(j):
                base = (j * 16 + lanes) * 16
                acc = plsc.load_gather(hist, [base])
                for l in range(1, 16):
                    acc = acc + plsc.load_gather(hist, [base + l])
                totals[pl.ds(j * 16, 16)] = acc

            def scn(i, carry):
                q, a, e, c = carry
                j = nreg - 1 - i
                v = totals[pl.ds(j * 16, 16)]
                incl = lax.rev(plsc.cumsum(lax.rev(v, (0,))), (0,)) + c
                excl = incl - v
                m = (excl < need) & (incl >= need)
                has = jnp.max(jnp.where(m, 1, 0))
                qq = j * 16 + jnp.max(jnp.where(m, lanes, 0))
                aa = jnp.max(jnp.where(m, excl, 0))
                ee = jnp.max(jnp.where(m, v, 0))
                q = jnp.where(has > 0, qq, q)
                a = jnp.where(has > 0, aa, a)
                e = jnp.where(has > 0, ee, e)
                return q, a, e, c + jnp.sum(v)
            q, a, e, _ = lax.fori_loop(0, nreg, scn, (0, 0, 0, 0))
            return q, a, e

        q0, a0, e0 = scan_hist(jnp.int32(NSEL), NB0 // 16)
        ok = (q0 > 0) & (a0 + e0 <= CAP)
        st[3] = jnp.where(ok, 1, 0)
        st[4] = K05 + lax.shift_left(q0, 14)
        st[5] = 0

        def level(shift, first, is_last):
            @pl.when(st[3] == 0)
            def _():
                if first:
                    prefix = jnp.int32(0)
                    need = jnp.int32(NSEL)
                    sure = jnp.int32(0)
                else:
                    prefix = st[0]
                    need = st[1]
                    sure = st[2]
                zero_hist(16)

                @plsc.parallel_loop(0, NVREG, unroll=8)
                def _pb(i):
                    key = data[pl.ds(i * 16, 16)]
                    if first:
                        d = lax.shift_right_arithmetic(key, 24) + 128
                        plsc.addupdate_scatter(
                            hist, [lax.shift_left(d, 4) | lanes], ones)
                    else:
                        m = lax.shift_right_arithmetic(key, shift + 8) == prefix
                        d = lax.shift_right_arithmetic(key, shift) & 255
                        plsc.addupdate_scatter(
                            hist, [lax.shift_left(d, 4) | lanes], ones, mask=m)

                q, a, e = scan_hist(need, 16)
                if first:
                    pref2 = q - 128
                else:
                    pref2 = lax.shift_left(prefix, 8) | q
                st[0] = pref2
                st[1] = need - a
                st[2] = sure + a
                cc = sure + a + e
                st[3] = 1 if is_last else jnp.where(cc <= CAP, 1, 0)
                st[5] = jnp.where(cc > CAP, 1, 0) if is_last else 0
                st[4] = lax.shift_left(pref2, shift)

        level(24, True, False)
        level(16, False, False)
        level(8, False, False)
        level(0, False, True)

        thr = st[4]
        exact = st[5]
        need_eq = st[1]

        @pl.when(exact == 0)
        def _():
            @plsc.parallel_loop(0, NVREG // 8, carry=jnp.int32(0))
            def c_a(i, off):
                ks = [data[pl.ds((i * 8 + u) * 16, 16)] for u in range(8)]
                sels = [k >= thr for k in ks]
                pcs = [plsc.all_reduce_population_count(s)[0] for s in sels]
                o = off
                for u in range(8):
                    plsc.store_compressed(ckey.at[pl.ds(o, 16)], ks[u],
                                          mask=sels[u])
                    plsc.store_compressed(cidx.at[pl.ds(o, 16)],
                                          (i * 8 + u) * 16 + lanes,
                                          mask=sels[u])
                    o = o + pcs[u]
                return o
            st[6] = c_a

        @pl.when(exact == 1)
        def _():
            def c_b(i, carry):
                off, taken = carry
                for u in range(4):
                    ii = i * 4 + u
                    key = data[pl.ds(ii * 16, 16)]
                    gt = key > thr
                    eq = key == thr
                    pceq = plsc.cumsum(jnp.where(eq, 1, 0))
                    seleq = eq & ((taken + pceq) <= need_eq)
                    sel = gt | seleq
                    plsc.store_compressed(ckey.at[pl.ds(off, 16)], key,
                                          mask=sel)
                    plsc.store_compressed(cidx.at[pl.ds(off, 16)],
                                          ii * 16 + lanes, mask=sel)
                    off = off + plsc.all_reduce_population_count(sel)[0]
                    taken = taken + plsc.all_reduce_population_count(seleq)[0]
                return off, taken
            cc, _t = lax.fori_loop(0, NVREG // 4, c_b, (0, 0))
            st[6] = cc

        nc = st[6]
        plsc.store_scatter(ckey, [nc + lanes], jnp.full((16,), IMIN, jnp.int32))
        plsc.store_scatter(cidx, [nc + lanes], zeros)

        nv = (nc + 15) // 16

        @plsc.parallel_loop(0, nv)
        def _rk(je):
            ke = ckey[pl.ds(je * 16, 16)]
            pose = je * 16 + lanes

            @plsc.parallel_loop(0, nv, carry=zeros)
            def rk_inner(jf, acc):
                base = jf * 16
                for r in range(16):
                    idx = base + ((lanes + r) & 15)
                    kv = plsc.load_gather(ckey, [idx])
                    hit = (kv > ke) | ((kv == ke) & (idx < pose))
                    acc = acc + jnp.where(hit, 1, 0)
                return acc
            crank[pl.ds(je * 16, 16)] = rk_inner

        tsx = tsv[pl.ds(2 * b, 16)]
        hf = tsx[0].astype(jnp.float32)
        wf = tsx[1].astype(jnp.float32)
        recip = jnp.float32(1.0 / NCLS)

        @plsc.parallel_loop(0, nv, unroll=2)
        def _post(j):
            key = ckey[pl.ds(j * 16, 16)]
            idxv = cidx[pl.ds(j * 16, 16)]
            rk = crank[pl.ds(j * 16, 16)]
            posv = j * 16 + lanes
            m = (rk < NSEL) & (posv < nc)
            score = plsc.bitcast(key, jnp.float32)
            qf = (idxv.astype(jnp.float32) * recip).astype(jnp.int32)
            r = idxv - qf * NCLS
            qq = qf + jnp.where(r >= NCLS, 1, 0) - jnp.where(r < 0, 1, 0)
            cls = idxv - qq * NCLS
            a0_ = lax.shift_left(qq, 2)
            cx = plsc.load_gather(boxv, [a0_], mask=m)
            cy = plsc.load_gather(boxv, [a0_ + 1], mask=m)
            w_ = plsc.load_gather(boxv, [a0_ + 2], mask=m)
            h_ = plsc.load_gather(boxv, [a0_ + 3], mask=m)
            x1 = (cx - 0.5 * w_) * wf
            y1 = (cy - 0.5 * h_) * hf
            x2 = (cx + 0.5 * w_) * wf
            y2 = (cy + 0.5 * h_) * hf
            plsc.store_scatter(stage_s, [rk], score, mask=m)
            plsc.store_scatter(stage_l, [rk], cls, mask=m)
            rb = lax.shift_left(rk, 2)
            plsc.store_scatter(stage_b, [rb], x1, mask=m)
            plsc.store_scatter(stage_b, [rb + 1], y1, mask=m)
            plsc.store_scatter(stage_b, [rb + 2], x2, mask=m)
            plsc.store_scatter(stage_b, [rb + 3], y2, mask=m)

        pltpu.sync_copy(stage_s, out_s_hbm.at[pl.ds(b * OPAD, OPAD)])
        pltpu.sync_copy(stage_l, out_l_hbm.at[pl.ds(b * OPAD, OPAD)])
        pltpu.sync_copy(stage_b, out_b_hbm.at[pl.ds(b * OPAD * 4, OPAD * 4)])


@jax.jit
def _sc_topk(prob_bits, boxes_flat, ts_flat):
    mesh = plsc.VectorSubcoreMesh(core_axis_name="c", subcore_axis_name="s",
                                  num_cores=2, num_subcores=16)
    fn = pl.kernel(
        _sc_body,
        out_type=(
            jax.ShapeDtypeStruct((NB * OPAD,), jnp.float32),
            jax.ShapeDtypeStruct((NB * OPAD,), jnp.int32),
            jax.ShapeDtypeStruct((NB * OPAD * 4,), jnp.float32),
        ),
        mesh=mesh,
        compiler_params=pltpu.CompilerParams(needs_layout_passes=False),
        scratch_types=[
            pltpu.VMEM((NPAD,), jnp.int32),
            pltpu.VMEM((4000,), jnp.float32),
            pltpu.VMEM((48,), jnp.int32),
            pltpu.VMEM((HISTW,), jnp.int32),
            pltpu.VMEM((NB0,), jnp.int32),
            pltpu.VMEM((CBUF,), jnp.int32),
            pltpu.VMEM((CBUF,), jnp.int32),
            pltpu.VMEM((CBUF,), jnp.int32),
            pltpu.VMEM((OPAD,), jnp.float32),
            pltpu.VMEM((OPAD,), jnp.int32),
            pltpu.VMEM((OPAD * 4,), jnp.float32),
            pltpu.SMEM((8,), jnp.int32),
        ],
    )
    return fn(prob_bits, boxes_flat, ts_flat)


def kernel(pred_logits, pred_boxes, target_sizes):
    B, N, C = pred_logits.shape
    prob = jax.nn.sigmoid(pred_logits.reshape(B * N * C))
    prob_bits = lax.bitcast_convert_type(prob, jnp.int32)
    scores_p, labels_p, boxes_p = _sc_topk(
        prob_bits, pred_boxes.reshape(-1), target_sizes.reshape(-1))
    scores = scores_p.reshape(NB, OPAD)[:, :NSEL]
    labels = labels_p.reshape(NB, OPAD)[:, :NSEL]
    boxes = boxes_p.reshape(NB, OPAD, 4)[:, :NSEL, :]
    return scores, labels, boxes


# speedup vs baseline: 2.6406x; 1.1072x over previous
import jax
import jax.numpy as jnp
from jax import lax
from jax.experimental import pallas as pl
from jax.experimental.pallas import tpu as pltpu
from jax.experimental.pallas import tpu_sc as plsc

NSEL = 300
NB = 16
NFLAT = 91000
NCLS = 91
CAP = 512
OPAD = 304
IMIN = -2147483648
K05 = 0x3F000000
NB0 = 544
HISTW = NB0 * 16

HOFF = 45496
NHALF = 45504
NLOC = 45568
NVH = NLOC // 16
EX = 624
CBUF2 = 1280

SROW = 4096
SH_TOT = 0
SH_KEY = 544
SH_IDX = 1168
SH_CNT = 1792
SH_S = 1808
SH_L = 2112
SH_B = 2416


def _sc_body(prob_hbm, boxes_hbm, ts_hbm,
             out_s_hbm, out_l_hbm, out_b_hbm,
             data, boxv, tsv, hist, totals, totals2,
             ckey, cidx, crank, stage_s, stage_l, stage_b, mrg, shr, st):
    cc_ = lax.axis_index("c")
    ss_ = lax.axis_index("s")
    b = cc_ * 8 + (ss_ // 2)
    h = ss_ % 2
    slot = ss_
    pslot = ss_ ^ 1

    lanes = lax.broadcasted_iota(jnp.int32, (16,), 0)
    zeros = jnp.zeros((16,), jnp.int32)
    ones = jnp.full((16,), 1, jnp.int32)
    gbase = h * HOFF

    pltpu.sync_copy(ts_hbm, tsv.at[pl.ds(0, 32)])
    pltpu.sync_copy(boxes_hbm.at[pl.ds(b * 4000, 4000)], boxv)
    pltpu.sync_copy(prob_hbm.at[pl.ds(b * NFLAT + gbase, NHALF)],
                    data.at[pl.ds(0, NHALF)])
    neg1 = jnp.full((16,), -1, jnp.int32)
    plsc.store_scatter(data, [lanes], neg1, mask=(lanes < 8) & (h == 1))
    for t in range(NHALF, NLOC, 16):
        data[pl.ds(t, 16)] = neg1

    def zero_hist(nreg):
        @plsc.parallel_loop(0, nreg, unroll=8)
        def _zh(j):
            hist[pl.ds(j * 16, 16)] = zeros

    zero_hist(NB0)

    @plsc.parallel_loop(0, NVH, unroll=8)
    def _pa(i):
        key = data[pl.ds(i * 16, 16)]
        d = lax.shift_right_logical(jnp.maximum(key, K05) - K05, 14)
        d = jnp.minimum(d, NB0 - 1)
        plsc.addupdate_scatter(hist, [lax.shift_left(d, 4) | lanes], ones)

    def totalize(nreg):
        @plsc.parallel_loop(0, nreg, unroll=2)
        def _tot(j):
            base = (j * 16 + lanes) * 16
            acc = plsc.load_gather(hist, [base])
            for l in range(1, 16):
                acc = acc + plsc.load_gather(hist, [base + l])
            totals[pl.ds(j * 16, 16)] = acc

    def merge_totals(nreg, active):
        @pl.when(active)
        def _():
            pltpu.sync_copy(totals.at[pl.ds(0, nreg * 16)],
                            shr.at[pl.ds(slot * SROW + SH_TOT, nreg * 16)])
        plsc.subcore_barrier()

        @pl.when(active)
        def _():
            pltpu.sync_copy(shr.at[pl.ds(pslot * SROW + SH_TOT, nreg * 16)],
                            totals2.at[pl.ds(0, nreg * 16)])
        plsc.subcore_barrier()

        @pl.when(active)
        def _():
            @plsc.parallel_loop(0, nreg, unroll=2)
            def _mg(j):
                totals[pl.ds(j * 16, 16)] = (totals[pl.ds(j * 16, 16)]
                                             + totals2[pl.ds(j * 16, 16)])

    def scan_cross(need, nreg):
        def scn(i, carry):
            q, a, e, c = carry
            j = nreg - 1 - i
            v = totals[pl.ds(j * 16, 16)]
            incl = lax.rev(plsc.cumsum(lax.rev(v, (0,))), (0,)) + c
            excl = incl - v
            m = (excl < need) & (incl >= need)
            has = jnp.max(jnp.where(m, 1, 0))
            qq = j * 16 + jnp.max(jnp.where(m, lanes, 0))
            aa = jnp.max(jnp.where(m, excl, 0))
            ee = jnp.max(jnp.where(m, v, 0))
            q = jnp.where(has > 0, qq, q)
            a = jnp.where(has > 0, aa, a)
            e = jnp.where(has > 0, ee, e)
            return q, a, e, c + jnp.sum(v)
        q, a, e, _ = lax.fori_loop(0, nreg, scn, (0, 0, 0, 0))
        return q, a, e

    totalize(NB0 // 16)
    merge_totals(NB0 // 16, True)
    q0, a0, e0 = scan_cross(jnp.int32(NSEL), NB0 // 16)
    ok = (q0 > 0) & (a0 + e0 <= CAP)
    st[3] = jnp.where(ok, 1, 0)
    st[4] = K05 + lax.shift_left(q0, 14)
    st[5] = 0

    def level(shift, first, is_last):
        active = st[3] == 0

        @pl.when(active)
        def _():
            zero_hist(16)

            @plsc.parallel_loop(0, NVH, unroll=8)
            def _pb(i):
                key = data[pl.ds(i * 16, 16)]
                if first:
                    d = lax.shift_right_arithmetic(key, 24) + 128
                    plsc.addupdate_scatter(
                        hist, [lax.shift_left(d, 4) | lanes], ones)
                else:
                    m = lax.shift_right_arithmetic(key, shift + 8) == st[0]
                    d = lax.shift_right_arithmetic(key, shift) & 255
                    plsc.addupdate_scatter(
                        hist, [lax.shift_left(d, 4) | lanes], ones, mask=m)
            totalize(16)
        merge_totals(16, active)

        @pl.when(active)
        def _():
            if first:
                prefix = jnp.int32(0)
                need = jnp.int32(NSEL)
                sure = jnp.int32(0)
            else:
                prefix = st[0]
                need = st[1]
                sure = st[2]
            q, a, e = scan_cross(need, 16)
            if first:
                pref2 = q - 128
            else:
                pref2 = lax.shift_left(prefix, 8) | q
            st[0] = pref2
            st[1] = need - a
            st[2] = sure + a
            cc = sure + a + e
            st[3] = 1 if is_last else jnp.where(cc <= CAP, 1, 0)
            st[5] = jnp.where(cc > CAP, 1, 0) if is_last else 0
            st[4] = lax.shift_left(pref2, shift)

    level(24, True, False)
    level(16, False, False)
    level(8, False, False)
    level(0, False, True)

    thr = st[4]
    exact = st[5]
    need_eq = st[1]

    @pl.when(exact == 0)
    def _():
        @plsc.parallel_loop(0, NVH // 8, carry=jnp.int32(0))
        def c_a(i, off):
            ks = [data[pl.ds((i * 8 + u) * 16, 16)] for u in range(8)]
            sels = [k >= thr for k in ks]
            pcs = [plsc.all_reduce_population_count(sl)[0] for sl in sels]
            o = off
            for u in range(8):
                plsc.store_compressed(ckey.at[pl.ds(o, 16)], ks[u],
                                      mask=sels[u])
                plsc.store_compressed(cidx.at[pl.ds(o, 16)],
                                      gbase + (i * 8 + u) * 16 + lanes,
                                      mask=sels[u])
                o = o + pcs[u]
            return o
        st[6] = c_a

    @pl.when(exact == 1)
    def _():
        def c_b(i, carry):
            off, taken = carry
            for u in range(4):
                ii = i * 4 + u
                key = data[pl.ds(ii * 16, 16)]
                gt = key > thr
                eq = key == thr
                pceq = plsc.cumsum(jnp.where(eq, 1, 0))
                seleq = eq & ((taken + pceq) <= need_eq)
                sel = gt | seleq
                plsc.store_compressed(ckey.at[pl.ds(off, 16)], key, mask=sel)
                plsc.store_compressed(cidx.at[pl.ds(off, 16)],
                                      gbase + ii * 16 + lanes, mask=sel)
                off = off + plsc.all_reduce_population_count(sel)[0]
                taken = taken + plsc.all_reduce_population_count(seleq)[0]
            return off, taken
        ccnt, _t = lax.fori_loop(0, NVH // 4, c_b, (0, 0))
        st[6] = ccnt

    nc = st[6]
    plsc.store_scatter(ckey, [nc + lanes], jnp.full((16,), IMIN, jnp.int32))
    plsc.store_scatter(cidx, [nc + lanes], zeros)

    tsv[pl.ds(32, 16)] = jnp.full((16,), nc)
    pltpu.sync_copy(ckey.at[pl.ds(0, EX)], shr.at[pl.ds(slot * SROW + SH_KEY, EX)])
    pltpu.sync_copy(cidx.at[pl.ds(0, EX)], shr.at[pl.ds(slot * SROW + SH_IDX, EX)])
    pltpu.sync_copy(tsv.at[pl.ds(32, 16)], shr.at[pl.ds(slot * SROW + SH_CNT, 16)])
    plsc.subcore_barrier()
    pltpu.sync_copy(shr.at[pl.ds(pslot * SROW + SH_KEY, EX)], ckey.at[pl.ds(EX, EX)])
    pltpu.sync_copy(shr.at[pl.ds(pslot * SROW + SH_IDX, EX)], cidx.at[pl.ds(EX, EX)])
    pltpu.sync_copy(shr.at[pl.ds(pslot * SROW + SH_CNT, 16)], tsv.at[pl.ds(32, 16)])
    plsc.subcore_barrier()
    nc2 = tsv[pl.ds(32, 16)][0]

    nv1 = (nc + 15) // 16
    nv2 = (nc2 + 15) // 16

    @plsc.parallel_loop(0, nv1)
    def _rk(je):
        ke = ckey[pl.ds(je * 16, 16)]
        ie = cidx[pl.ds(je * 16, 16)]

        def other(jf, acc, base):
            for r in range(16):
                idx = base + jf * 16 + ((lanes + r) & 15)
                kv = plsc.load_gather(ckey, [idx])
                iv = plsc.load_gather(cidx, [idx])
                hit = (kv > ke) | ((kv == ke) & (iv < ie))
                acc = acc + jnp.where(hit, 1, 0)
            return acc

        @plsc.parallel_loop(0, nv1, carry=zeros)
        def acc_own(jf, acc):
            return other(jf, acc, 0)

        @plsc.parallel_loop(0, nv2, carry=acc_own)
        def acc_all(jf, acc):
            return other(jf, acc, EX)

        crank[pl.ds(je * 16, 16)] = acc_all

    tsx = tsv[pl.ds(2 * b, 16)]
    hf = tsx[0].astype(jnp.float32)
    wf = tsx[1].astype(jnp.float32)
    recip = jnp.float32(1.0 / NCLS)

    @plsc.parallel_loop(0, OPAD // 16, unroll=2)
    def _zs(j):
        stage_s[pl.ds(j * 16, 16)] = zeros
        stage_l[pl.ds(j * 16, 16)] = zeros
        for u in range(4):
            stage_b[pl.ds(j * 64 + u * 16, 16)] = zeros

    @plsc.parallel_loop(0, nv1, unroll=2)
    def _post(j):
        key = ckey[pl.ds(j * 16, 16)]
        idxv = cidx[pl.ds(j * 16, 16)]
        rk = crank[pl.ds(j * 16, 16)]
        posv = j * 16 + lanes
        m = (rk < NSEL) & (posv < nc)
        qf = (idxv.astype(jnp.float32) * recip).astype(jnp.int32)
        r = idxv - qf * NCLS
        qq = qf + jnp.where(r >= NCLS, 1, 0) - jnp.where(r < 0, 1, 0)
        cls = idxv - qq * NCLS
        a0_ = lax.shift_left(qq, 2)
        cx = plsc.load_gather(boxv, [a0_], mask=m)
        cy = plsc.load_gather(boxv, [a0_ + 1], mask=m)
        w_ = plsc.load_gather(boxv, [a0_ + 2], mask=m)
        h_ = plsc.load_gather(boxv, [a0_ + 3], mask=m)
        x1 = plsc.bitcast((cx - 0.5 * w_) * wf, jnp.int32)
        y1 = plsc.bitcast((cy - 0.5 * h_) * hf, jnp.int32)
        x2 = plsc.bitcast((cx + 0.5 * w_) * wf, jnp.int32)
        y2 = plsc.bitcast((cy + 0.5 * h_) * hf, jnp.int32)
        plsc.store_scatter(stage_s, [rk], key, mask=m)
        plsc.store_scatter(stage_l, [rk], cls, mask=m)
        rb = lax.shift_left(rk, 2)
        plsc.store_scatter(stage_b, [rb], x1, mask=m)
        plsc.store_scatter(stage_b, [rb + 1], y1, mask=m)
        plsc.store_scatter(stage_b, [rb + 2], x2, mask=m)
        plsc.store_scatter(stage_b, [rb + 3], y2, mask=m)

    pltpu.sync_copy(stage_s, shr.at[pl.ds(slot * SROW + SH_S, OPAD)])
    pltpu.sync_copy(stage_l, shr.at[pl.ds(slot * SROW + SH_L, OPAD)])
    pltpu.sync_copy(stage_b, shr.at[pl.ds(slot * SROW + SH_B, OPAD * 4)])
    plsc.subcore_barrier()

    @pl.when(h == 0)
    def _():
        pltpu.sync_copy(shr.at[pl.ds(pslot * SROW + SH_S, OPAD)], mrg.at[pl.ds(0, OPAD)])

        @plsc.parallel_loop(0, OPAD // 16, unroll=2)
        def _m1(j):
            stage_s[pl.ds(j * 16, 16)] = (stage_s[pl.ds(j * 16, 16)]
                                          + mrg[pl.ds(j * 16, 16)])
        pltpu.sync_copy(shr.at[pl.ds(pslot * SROW + SH_L, OPAD)], mrg.at[pl.ds(0, OPAD)])

        @plsc.parallel_loop(0, OPAD // 16, unroll=2)
        def _m2(j):
            stage_l[pl.ds(j * 16, 16)] = (stage_l[pl.ds(j * 16, 16)]
                                          + mrg[pl.ds(j * 16, 16)])
        pltpu.sync_copy(shr.at[pl.ds(pslot * SROW + SH_B, OPAD * 4)], mrg)

        @plsc.parallel_loop(0, OPAD * 4 // 16, unroll=2)
        def _m3(j):
            stage_b[pl.ds(j * 16, 16)] = (stage_b[pl.ds(j * 16, 16)]
                                          + mrg[pl.ds(j * 16, 16)])
        pltpu.sync_copy(stage_s, out_s_hbm.at[pl.ds(b * OPAD, OPAD)])
        pltpu.sync_copy(stage_l, out_l_hbm.at[pl.ds(b * OPAD, OPAD)])
        pltpu.sync_copy(stage_b, out_b_hbm.at[pl.ds(b * OPAD * 4, OPAD * 4)])


@jax.jit
def _sc_topk(prob_bits, boxes_flat, ts_flat):
    mesh = plsc.VectorSubcoreMesh(core_axis_name="c", subcore_axis_name="s",
                                  num_cores=2, num_subcores=16)
    fn = pl.kernel(
        _sc_body,
        out_type=(
            jax.ShapeDtypeStruct((NB * OPAD,), jnp.int32),
            jax.ShapeDtypeStruct((NB * OPAD,), jnp.int32),
            jax.ShapeDtypeStruct((NB * OPAD * 4,), jnp.int32),
        ),
        mesh=mesh,
        compiler_params=pltpu.CompilerParams(needs_layout_passes=False),
        scratch_types=[
            pltpu.VMEM((NLOC,), jnp.int32),
            pltpu.VMEM((4000,), jnp.float32),
            pltpu.VMEM((48,), jnp.int32),
            pltpu.VMEM((HISTW,), jnp.int32),
            pltpu.VMEM((NB0,), jnp.int32),
            pltpu.VMEM((NB0,), jnp.int32),
            pltpu.VMEM((CBUF2,), jnp.int32),
            pltpu.VMEM((CBUF2,), jnp.int32),
            pltpu.VMEM((EX + 16,), jnp.int32),
            pltpu.VMEM((OPAD,), jnp.int32),
            pltpu.VMEM((OPAD,), jnp.int32),
            pltpu.VMEM((OPAD * 4,), jnp.int32),
            pltpu.VMEM((OPAD * 4,), jnp.int32),
            pltpu.VMEM_SHARED((16 * SROW,), jnp.int32),
            pltpu.SMEM((8,), jnp.int32),
        ],
    )
    return fn(prob_bits, boxes_flat, ts_flat)


def kernel(pred_logits, pred_boxes, target_sizes):
    B, N, C = pred_logits.shape
    prob = jax.nn.sigmoid(pred_logits.reshape(B * N * C))
    prob_bits = lax.bitcast_convert_type(prob, jnp.int32)
    scores_p, labels_p, boxes_p = _sc_topk(
        prob_bits, pred_boxes.reshape(-1), target_sizes.reshape(-1))
    scores = lax.bitcast_convert_type(
        scores_p, jnp.float32).reshape(NB, OPAD)[:, :NSEL]
    labels = labels_p.reshape(NB, OPAD)[:, :NSEL]
    boxes = lax.bitcast_convert_type(
        boxes_p, jnp.float32).reshape(NB, OPAD, 4)[:, :NSEL, :]
    return scores, labels, boxes
